# Initial kernel scaffold; baseline (speedup 1.0000x reference)
import functools

import jax
import jax.numpy as jnp
from jax import lax
from jax.experimental import pallas as pl
from jax.experimental.pallas import tpu as pltpu
from jax.experimental.pallas import tpu_sc as plsc

HID = 128
B = 1024
S = 200
SPAD = 208
NA = 112
NB = 96
LANES = 16

_NC = 2
_NS = 16
NW = _NC * _NS
ROWS_PER_W = B // NW


def _sc_gather_sum(ids_flat, word, pos):
    mesh = plsc.VectorSubcoreMesh(core_axis_name="c", subcore_axis_name="s")

    @functools.partial(
        pl.kernel,
        out_type=jax.ShapeDtypeStruct((B * S, HID), jnp.float32),
        mesh=mesh,
        scratch_types=[
            pltpu.VMEM((NA,), jnp.int32),
            pltpu.VMEM((NB,), jnp.int32),
            pltpu.VMEM((NA,), jnp.int32),
            pltpu.VMEM((NB,), jnp.int32),
            pltpu.VMEM((SPAD, HID), jnp.float32),
            pltpu.VMEM((SPAD, HID), jnp.float32),
            pltpu.SemaphoreType.DMA,
        ],
    )
    def k(ids_hbm, word_hbm, pos_hbm, out_hbm, ida, idb, pida, pidb, wbuf, pbuf, sem):
        wid = lax.axis_index("s") * _NC + lax.axis_index("c")

        def row_body(i, carry_unused):
            row = wid * ROWS_PER_W + i
            base = row * S
            pltpu.sync_copy(ids_hbm.at[pl.ds(base, NA)], ida)
            pltpu.sync_copy(ids_hbm.at[pl.ds(base + NA, S - NA)],
                            idb.at[pl.ds(0, S - NA)])

            lane = lax.iota(jnp.int32, LANES)
            carry = jnp.int32(0)
            for c in range(SPAD // LANES):
                if c < NA // LANES:
                    src, off = ida, c * LANES
                    psrc = pida
                else:
                    src, off = idb, c * LANES - NA
                    psrc = pidb
                v = src[pl.ds(off, LANES)]
                if c == SPAD // LANES - 1:
                    v = jnp.where(lane < (S - (SPAD - LANES)), v, 0)
                    src[pl.ds(off, LANES)] = v
                m = (v != 0).astype(jnp.int32)
                cs = plsc.cumsum(m)
                psrc[pl.ds(off, LANES)] = (cs + carry) * m
                carry = carry + jnp.sum(m)

            c1 = pltpu.async_copy(word_hbm.at[ida], wbuf.at[pl.ds(0, NA)], sem)
            c2 = pltpu.async_copy(word_hbm.at[idb], wbuf.at[pl.ds(NA, NB)], sem)
            c3 = pltpu.async_copy(pos_hbm.at[pida], pbuf.at[pl.ds(0, NA)], sem)
            c4 = pltpu.async_copy(pos_hbm.at[pidb], pbuf.at[pl.ds(NA, NB)], sem)
            c1.wait(); c2.wait(); c3.wait(); c4.wait()

            def add_body(t, carry2):
                for j in range(HID // LANES):
                    wbuf[t, pl.ds(j * LANES, LANES)] = (
                        wbuf[t, pl.ds(j * LANES, LANES)]
                        + pbuf[t, pl.ds(j * LANES, LANES)])
                return carry2
            lax.fori_loop(0, S, add_body, 0)

            pltpu.sync_copy(wbuf.at[pl.ds(0, S)], out_hbm.at[pl.ds(base, S)])
            return carry_unused

        lax.fori_loop(0, ROWS_PER_W, row_body, 0)

    return k(ids_flat, word, pos)


def _tc_layernorm(x, typ0, gamma, beta):
    ROWS = 2048
    n_blocks = (B * S) // ROWS

    def body(x_ref, t_ref, g_ref, b_ref, o_ref):
        x = x_ref[...] + t_ref[...]
        mu = jnp.mean(x, axis=-1, keepdims=True)
        xc = x - mu
        var = jnp.mean(xc * xc, axis=-1, keepdims=True)
        o_ref[...] = xc * lax.rsqrt(var + 1e-12) * g_ref[...] + b_ref[...]

    return pl.pallas_call(
        body,
        grid=(n_blocks,),
        in_specs=[
            pl.BlockSpec((ROWS, HID), lambda i: (i, 0)),
            pl.BlockSpec((1, HID), lambda i: (0, 0)),
            pl.BlockSpec((1, HID), lambda i: (0, 0)),
            pl.BlockSpec((1, HID), lambda i: (0, 0)),
        ],
        out_specs=pl.BlockSpec((ROWS, HID), lambda i: (i, 0)),
        out_shape=jax.ShapeDtypeStruct((B * S, HID), jnp.float32),
    )(x, typ0, gamma, beta)


def kernel(input_ids, word_embeddings, position_embeddings,
           token_type_embeddings, ln_weight, ln_bias):
    ids_flat = input_ids.astype(jnp.int32).reshape(B * S)
    sums = _sc_gather_sum(ids_flat, word_embeddings, position_embeddings)
    typ0 = token_type_embeddings[0:1]
    out = _tc_layernorm(sums, typ0,
                        ln_weight.reshape(1, HID), ln_bias.reshape(1, HID))
    return out.reshape(B, S, HID)


# speedup vs baseline: 3.4941x; 3.4941x over previous
import functools

import jax
import jax.numpy as jnp
from jax import lax
from jax.experimental import pallas as pl
from jax.experimental.pallas import tpu as pltpu
from jax.experimental.pallas import tpu_sc as plsc

HID = 128
B = 1024
S = 200
SPAD = 208
NA = 112
NB = 96
LANES = 16

_NC = 2
_NS = 16
NW = _NC * _NS
ROWS_PER_W = B // NW


def _sc_gather_sum(ids_flat, word, pos):
    mesh = plsc.VectorSubcoreMesh(core_axis_name="c", subcore_axis_name="s")

    @functools.partial(
        pl.kernel,
        out_type=jax.ShapeDtypeStruct((B * S, HID), jnp.float32),
        mesh=mesh,
        scratch_types=[
            pltpu.VMEM((NA,), jnp.int32),
            pltpu.VMEM((NB,), jnp.int32),
            pltpu.VMEM((NA,), jnp.int32),
            pltpu.VMEM((NB,), jnp.int32),
            pltpu.VMEM((SPAD, HID), jnp.float32),
            pltpu.VMEM((SPAD, HID), jnp.float32),
            pltpu.SemaphoreType.DMA,
        ],
        compiler_params=pltpu.CompilerParams(needs_layout_passes=False),
    )
    def k(ids_hbm, word_hbm, pos_hbm, out_hbm, ida, idb, pida, pidb, wbuf, pbuf, sem):
        wid = lax.axis_index("s") * _NC + lax.axis_index("c")

        def row_body(i, carry_unused):
            row = wid * ROWS_PER_W + i
            base = row * S
            pltpu.sync_copy(ids_hbm.at[pl.ds(base, NA)], ida)
            pltpu.sync_copy(ids_hbm.at[pl.ds(base + NA, S - NA)],
                            idb.at[pl.ds(0, S - NA)])

            lane = lax.iota(jnp.int32, LANES)
            ntail = S - (SPAD - LANES)
            tailmask = lax.shift_right_logical(
                (ntail - 1) - lane + 16, jnp.int32(4)
            ) & 1
            carry = jnp.int32(0)
            for c in range(SPAD // LANES):
                if c < NA // LANES:
                    src, off = ida, c * LANES
                    psrc = pida
                else:
                    src, off = idb, c * LANES - NA
                    psrc = pidb
                v = src[pl.ds(off, LANES)]
                if c == SPAD // LANES - 1:
                    v = v * tailmask
                    src[pl.ds(off, LANES)] = v
                m = jnp.minimum(jnp.abs(v), 1)
                cs = plsc.cumsum(m)
                psrc[pl.ds(off, LANES)] = (cs + carry) * m
                carry = carry + jnp.sum(m)

            c1 = pltpu.async_copy(word_hbm.at[ida], wbuf.at[pl.ds(0, NA)], sem)
            c2 = pltpu.async_copy(word_hbm.at[idb], wbuf.at[pl.ds(NA, NB)], sem)
            c3 = pltpu.async_copy(pos_hbm.at[pida], pbuf.at[pl.ds(0, NA)], sem)
            c4 = pltpu.async_copy(pos_hbm.at[pidb], pbuf.at[pl.ds(NA, NB)], sem)
            c1.wait(); c2.wait(); c3.wait(); c4.wait()

            def add_body(t, carry2):
                for j in range(HID // LANES):
                    wbuf[t, pl.ds(j * LANES, LANES)] = (
                        wbuf[t, pl.ds(j * LANES, LANES)]
                        + pbuf[t, pl.ds(j * LANES, LANES)])
                return carry2
            lax.fori_loop(0, S, add_body, 0)

            pltpu.sync_copy(wbuf.at[pl.ds(0, S)], out_hbm.at[pl.ds(base, S)])
            return carry_unused

        lax.fori_loop(0, ROWS_PER_W, row_body, 0)

    return k(ids_flat, word, pos)


def _tc_layernorm(x, typ0, gamma, beta):
    ROWS = 2048
    n_blocks = (B * S) // ROWS

    def body(x_ref, t_ref, g_ref, b_ref, o_ref):
        x = x_ref[...] + t_ref[...]
        mu = jnp.mean(x, axis=-1, keepdims=True)
        xc = x - mu
        var = jnp.mean(xc * xc, axis=-1, keepdims=True)
        o_ref[...] = xc * lax.rsqrt(var + 1e-12) * g_ref[...] + b_ref[...]

    return pl.pallas_call(
        body,
        grid=(n_blocks,),
        in_specs=[
            pl.BlockSpec((ROWS, HID), lambda i: (i, 0)),
            pl.BlockSpec((1, HID), lambda i: (0, 0)),
            pl.BlockSpec((1, HID), lambda i: (0, 0)),
            pl.BlockSpec((1, HID), lambda i: (0, 0)),
        ],
        out_specs=pl.BlockSpec((ROWS, HID), lambda i: (i, 0)),
        out_shape=jax.ShapeDtypeStruct((B * S, HID), jnp.float32),
    )(x, typ0, gamma, beta)


def kernel(input_ids, word_embeddings, position_embeddings,
           token_type_embeddings, ln_weight, ln_bias):
    ids_flat = input_ids.astype(jnp.int32).reshape(B * S)
    sums = _sc_gather_sum(ids_flat, word_embeddings, position_embeddings)
    typ0 = token_type_embeddings[0:1]
    out = _tc_layernorm(sums, typ0,
                        ln_weight.reshape(1, HID), ln_bias.reshape(1, HID))
    return out.reshape(B, S, HID)
